# baseline (speedup 1.0000x reference)
import functools

import jax
import jax.numpy as jnp
from jax import lax
from jax.experimental import pallas as pl
from jax.experimental.pallas import tpu as pltpu
from jax.experimental.pallas import tpu_sc as plsc

_K = 20
_KPAD = 32
_CHUNK = 512
_NC = 2
_NS = 16
_NW = _NC * _NS


def _leaky(h):
    return jnp.where(h >= 0, h, 0.2 * h)


def _topk_kernel(x_ref, amb_ref, bt_ref, b1_ref, idx_ref, u_ref, v_ref, neg_ref):
    n = x_ref.shape[2]
    chunk = u_ref.shape[0]
    b = pl.program_id(0)
    c0 = pl.program_id(1) * chunk

    xf = x_ref[0]
    xx = jnp.sum(xf * xf, axis=0, keepdims=True)
    xc = x_ref[0, :, pl.ds(c0, chunk)].T

    mm = jnp.dot(xc, xf, preferred_element_type=jnp.float32)
    xx_r = jnp.sum(xc * xc, axis=1, keepdims=True)
    neg_ref[...] = (2.0 * mm - xx_r) - xx

    u_ref[...] = jnp.dot(xc, amb_ref[...], preferred_element_type=jnp.float32) \
        + b1_ref[0:1, :]
    v_ref[...] = jnp.dot(xc, bt_ref[...], preferred_element_type=jnp.float32)

    iota = jax.lax.broadcasted_iota(jnp.int32, (chunk, n), 1)
    kiota = jax.lax.broadcasted_iota(jnp.int32, (chunk, _KPAD), 1)

    def body(k, carry):
        m, idxacc = carry
        d = neg_ref[...]
        idx = jnp.min(jnp.where(d == m, iota, n), axis=1, keepdims=True)
        onehot = iota == idx
        dn = jnp.where(onehot, -jnp.inf, d)
        neg_ref[...] = dn
        m = jnp.max(dn, axis=1, keepdims=True)
        return m, jnp.where(kiota == k, idx + b * n, idxacc)

    m0 = jnp.max(neg_ref[...], axis=1, keepdims=True)
    idxacc = jnp.zeros((chunk, _KPAD), dtype=jnp.int32)
    _, idxacc = jax.lax.fori_loop(0, _K, body, (m0, idxacc))
    idx_ref[...] = idxacc.T


def _mlp_kernel(u_ref, w2t_ref, b2_ref, *refs):
    g_refs = refs[:_K]
    out_ref = refs[_K]
    u = u_ref[...]
    w2t = w2t_ref[...]
    b2 = b2_ref[0:1, :]
    acc = jnp.full(u.shape, -jnp.inf, dtype=jnp.float32)
    for k in range(_K):
        h = _leaky(u + g_refs[k][:, :64])
        h = _leaky(jnp.dot(h, w2t, preferred_element_type=jnp.float32) + b2)
        acc = jnp.maximum(acc, h)
    out_ref[0] = acc.T


def _make_sc_gather(bn):
    span = bn // _NW
    half = span // 2
    mesh = plsc.VectorSubcoreMesh(core_axis_name="c", subcore_axis_name="s")

    @functools.partial(
        pl.kernel, mesh=mesh,
        out_type=jax.ShapeDtypeStruct((_K * bn, 128), jnp.float32),
        scratch_types=[
            pltpu.VMEM((half,), jnp.int32),
            pltpu.VMEM((half, 128), jnp.float32),
            pltpu.SemaphoreType.DMA,
        ],
        compiler_params=pltpu.CompilerParams(use_tc_tiling_on_sc=True),
    )
    def gather(idx_hbm, v_hbm, g_hbm, idx_v, rows_v, sem):
        wid = lax.axis_index("s") * _NC + lax.axis_index("c")
        base = wid * span

        def step(t, carry):
            k = t // 2
            off = base + (t % 2) * half
            pltpu.sync_copy(idx_hbm.at[k, pl.ds(off, half)], idx_v)
            pltpu.async_copy(v_hbm.at[idx_v], rows_v, sem).wait()
            pltpu.sync_copy(rows_v, g_hbm.at[pl.ds(k * bn + off, half)])
            return carry

        lax.fori_loop(0, 2 * _K, step, 0)

    return gather


@jax.jit
def kernel(x, W1, b1, W2, b2):
    B, C, N = x.shape
    bn = B * N
    nc = N // _CHUNK
    xp = jnp.pad(x, ((0, 0), (0, 8 - C), (0, 0)))
    A = W1[:, :C]
    Bm = W1[:, C:]
    amb = jnp.pad((A - Bm).T, ((0, 8 - C), (0, 0)))
    bt = jnp.pad(Bm.T, ((0, 8 - C), (0, 64)))
    b1b = jnp.broadcast_to(b1[None, :], (8, 64))
    b2b = jnp.broadcast_to(b2[None, :], (8, 64))

    idx32, u, v = pl.pallas_call(
        _topk_kernel,
        grid=(B, nc),
        in_specs=[
            pl.BlockSpec((1, 8, N), lambda b, c: (b, 0, 0)),
            pl.BlockSpec((8, 64), lambda b, c: (0, 0)),
            pl.BlockSpec((8, 128), lambda b, c: (0, 0)),
            pl.BlockSpec((8, 64), lambda b, c: (0, 0)),
        ],
        out_specs=[
            pl.BlockSpec((_KPAD, _CHUNK), lambda b, c: (0, b * (N // _CHUNK) + c)),
            pl.BlockSpec((_CHUNK, 64), lambda b, c: (b * (N // _CHUNK) + c, 0)),
            pl.BlockSpec((_CHUNK, 128), lambda b, c: (b * (N // _CHUNK) + c, 0)),
        ],
        out_shape=[
            jax.ShapeDtypeStruct((_KPAD, bn), jnp.int32),
            jax.ShapeDtypeStruct((bn, 64), jnp.float32),
            jax.ShapeDtypeStruct((bn, 128), jnp.float32),
        ],
        scratch_shapes=[pltpu.VMEM((_CHUNK, N), jnp.float32)],
        compiler_params=pltpu.CompilerParams(
            dimension_semantics=("parallel", "parallel")),
    )(xp, amb, bt, b1b)

    g = _make_sc_gather(bn)(idx32, v)

    g_specs = [
        pl.BlockSpec((N, 128), (lambda k: (lambda b: (k * B + b, 0)))(k))
        for k in range(_K)
    ]
    out = pl.pallas_call(
        _mlp_kernel,
        grid=(B,),
        in_specs=[
            pl.BlockSpec((N, 64), lambda b: (b, 0)),
            pl.BlockSpec((64, 64), lambda b: (0, 0)),
            pl.BlockSpec((8, 64), lambda b: (0, 0)),
        ] + g_specs,
        out_specs=pl.BlockSpec((1, 64, N), lambda b: (b, 0, 0)),
        out_shape=jax.ShapeDtypeStruct((B, 64, N), jnp.float32),
        compiler_params=pltpu.CompilerParams(
            dimension_semantics=("parallel",)),
    )(u, W2.T, b2b, *[g] * _K)
    return out


# speedup vs baseline: 1.0309x; 1.0309x over previous
import functools

import jax
import jax.numpy as jnp
from jax import lax
from jax.experimental import pallas as pl
from jax.experimental.pallas import tpu as pltpu
from jax.experimental.pallas import tpu_sc as plsc

_K = 20
_KPAD = 32
_CHUNK = 1024
_NC = 2
_NS = 16
_NW = _NC * _NS


def _leaky(h):
    return jnp.where(h >= 0, h, 0.2 * h)


def _topk_kernel(x_ref, amb_ref, bt_ref, b1_ref, idx_ref, u_ref, v_ref, neg_ref):
    n = x_ref.shape[2]
    chunk = u_ref.shape[0]
    b = pl.program_id(0)
    c0 = pl.program_id(1) * chunk

    xf = x_ref[0]
    xx = jnp.sum(xf * xf, axis=0, keepdims=True)
    xc = x_ref[0, :, pl.ds(c0, chunk)].T

    mm = jnp.dot(xc, xf, preferred_element_type=jnp.float32)
    xx_r = jnp.sum(xc * xc, axis=1, keepdims=True)
    neg_ref[...] = (2.0 * mm - xx_r) - xx

    u_ref[...] = jnp.dot(xc, amb_ref[...], preferred_element_type=jnp.float32) \
        + b1_ref[0:1, :]
    v_ref[...] = jnp.dot(xc, bt_ref[...], preferred_element_type=jnp.float32)

    iota = jax.lax.broadcasted_iota(jnp.int32, (chunk, n), 1)
    kiota = jax.lax.broadcasted_iota(jnp.int32, (chunk, _KPAD), 1)

    def body(k, idxacc):
        d = neg_ref[...]
        m = jnp.max(d, axis=1, keepdims=True)
        idx = jnp.min(jnp.where(d == m, iota, n), axis=1, keepdims=True)
        onehot = iota == idx
        neg_ref[...] = jnp.where(onehot, -jnp.inf, d)
        return jnp.where(kiota == k, idx + b * n, idxacc)

    idxacc = jnp.zeros((chunk, _KPAD), dtype=jnp.int32)
    idxacc = jax.lax.fori_loop(0, _K, body, idxacc)
    idx_ref[...] = idxacc.T


def _mlp_kernel(u_ref, w2t_ref, b2_ref, *refs):
    g_refs = refs[:_K]
    out_ref = refs[_K]
    u = u_ref[...]
    w2t = w2t_ref[...]
    b2 = b2_ref[0:1, :]
    acc = jnp.full(u.shape, -jnp.inf, dtype=jnp.float32)
    for k in range(_K):
        h = _leaky(u + g_refs[k][:, :64])
        h = _leaky(jnp.dot(h, w2t, preferred_element_type=jnp.float32) + b2)
        acc = jnp.maximum(acc, h)
    out_ref[0] = acc.T


def _make_sc_gather(bn):
    span = bn // _NW
    half = span // 2
    mesh = plsc.VectorSubcoreMesh(core_axis_name="c", subcore_axis_name="s")

    @functools.partial(
        pl.kernel, mesh=mesh,
        out_type=jax.ShapeDtypeStruct((_K * bn, 128), jnp.float32),
        scratch_types=[
            pltpu.VMEM((half,), jnp.int32),
            pltpu.VMEM((half, 128), jnp.float32),
            pltpu.SemaphoreType.DMA,
        ],
        compiler_params=pltpu.CompilerParams(use_tc_tiling_on_sc=True),
    )
    def gather(idx_hbm, v_hbm, g_hbm, idx_v, rows_v, sem):
        wid = lax.axis_index("s") * _NC + lax.axis_index("c")
        base = wid * span

        def step(t, carry):
            k = t // 2
            off = base + (t % 2) * half
            pltpu.sync_copy(idx_hbm.at[k, pl.ds(off, half)], idx_v)
            pltpu.async_copy(v_hbm.at[idx_v], rows_v, sem).wait()
            pltpu.sync_copy(rows_v, g_hbm.at[pl.ds(k * bn + off, half)])
            return carry

        lax.fori_loop(0, 2 * _K, step, 0)

    return gather


@jax.jit
def kernel(x, W1, b1, W2, b2):
    B, C, N = x.shape
    bn = B * N
    nc = N // _CHUNK
    xp = jnp.pad(x, ((0, 0), (0, 8 - C), (0, 0)))
    A = W1[:, :C]
    Bm = W1[:, C:]
    amb = jnp.pad((A - Bm).T, ((0, 8 - C), (0, 0)))
    bt = jnp.pad(Bm.T, ((0, 8 - C), (0, 64)))
    b1b = jnp.broadcast_to(b1[None, :], (8, 64))
    b2b = jnp.broadcast_to(b2[None, :], (8, 64))

    idx32, u, v = pl.pallas_call(
        _topk_kernel,
        grid=(B, nc),
        in_specs=[
            pl.BlockSpec((1, 8, N), lambda b, c: (b, 0, 0)),
            pl.BlockSpec((8, 64), lambda b, c: (0, 0)),
            pl.BlockSpec((8, 128), lambda b, c: (0, 0)),
            pl.BlockSpec((8, 64), lambda b, c: (0, 0)),
        ],
        out_specs=[
            pl.BlockSpec((_KPAD, _CHUNK), lambda b, c: (0, b * (N // _CHUNK) + c)),
            pl.BlockSpec((_CHUNK, 64), lambda b, c: (b * (N // _CHUNK) + c, 0)),
            pl.BlockSpec((_CHUNK, 128), lambda b, c: (b * (N // _CHUNK) + c, 0)),
        ],
        out_shape=[
            jax.ShapeDtypeStruct((_KPAD, bn), jnp.int32),
            jax.ShapeDtypeStruct((bn, 64), jnp.float32),
            jax.ShapeDtypeStruct((bn, 128), jnp.float32),
        ],
        scratch_shapes=[pltpu.VMEM((_CHUNK, N), jnp.float32)],
        compiler_params=pltpu.CompilerParams(
            dimension_semantics=("parallel", "parallel")),
    )(xp, amb, bt, b1b)

    g = _make_sc_gather(bn)(idx32, v)

    g_specs = [
        pl.BlockSpec((N, 128), (lambda k: (lambda b: (k * B + b, 0)))(k))
        for k in range(_K)
    ]
    out = pl.pallas_call(
        _mlp_kernel,
        grid=(B,),
        in_specs=[
            pl.BlockSpec((N, 64), lambda b: (b, 0)),
            pl.BlockSpec((64, 64), lambda b: (0, 0)),
            pl.BlockSpec((8, 64), lambda b: (0, 0)),
        ] + g_specs,
        out_specs=pl.BlockSpec((1, 64, N), lambda b: (b, 0, 0)),
        out_shape=jax.ShapeDtypeStruct((B, 64, N), jnp.float32),
        compiler_params=pltpu.CompilerParams(
            dimension_semantics=("parallel",)),
    )(u, W2.T, b2b, *[g] * _K)
    return out
